# baseline (speedup 1.0000x reference)
import functools

import jax
import jax.numpy as jnp
from jax import lax
from jax.experimental import pallas as pl
from jax.experimental.pallas import tpu as pltpu
from jax.experimental.pallas import tpu_sc as plsc

NC = 2
NS = 16
NW = NC * NS
LANES = 16


def _mesh():
    return plsc.VectorSubcoreMesh(
        core_axis_name="c", subcore_axis_name="s",
        num_cores=NC, num_subcores=NS)


@functools.lru_cache(maxsize=None)
def _build(B, F, N, K, interpret=False):
    assert K == LANES
    assert B % NW == 0 and N % 8 == 0
    S = B // NW
    C = 16
    assert S % C == 0
    NCH = S // C
    RPC = C * F
    GSZ = 128
    while RPC % GSZ or GSZ % 8:
        GSZ -= 8
    NSTR = RPC // GSZ
    NBUF = 2

    @functools.partial(
        pl.kernel,
        out_type=jax.ShapeDtypeStruct((B,), jnp.float32),
        mesh=_mesh(),
        scratch_types=[
            pltpu.VMEM((S * F,), jnp.int32),
            pltpu.VMEM((NBUF, RPC), jnp.int32),
            pltpu.VMEM((NBUF, RPC, 128), jnp.float32),
            pltpu.VMEM((NBUF, RPC), jnp.float32),
            pltpu.VMEM((S,), jnp.float32),
            pltpu.SemaphoreType.DMA((NBUF,)),
            pltpu.SemaphoreType.DMA((NBUF,)),
        ],
        compiler_params=pltpu.CompilerParams(
            needs_layout_passes=False, use_tc_tiling_on_sc=False),
        interpret=interpret,
    )
    def fm(idx_hbm, w_hbm, v8_hbm, out_hbm, idx_v, idx8_v, rows_v, wv_v,
           out_v, sem_v, sem_w):
        wid = lax.axis_index("s") * NC + lax.axis_index("c")
        base = wid * (S * F)
        pltpu.sync_copy(idx_hbm.at[pl.ds(base, S * F)], idx_v)

        lane = lax.iota(jnp.int32, LANES)
        lane_f = lane * F
        last = lane == (LANES - 1)

        def _cb(g):
            return g * RPC if isinstance(g, int) else pl.multiple_of(g * RPC, RPC)

        def prep_and_fire(g, slot):
            cb = _cb(g)
            for j in range(RPC // LANES):
                iv = idx_v[pl.ds(cb + j * LANES, LANES)]
                idx8_v[slot, pl.ds(j * LANES, LANES)] = (
                    lax.shift_right_logical(iv, 3))
            for j in range(NSTR):
                i8sl = idx8_v.at[slot, pl.ds(j * GSZ, GSZ)]
                pltpu.async_copy(
                    v8_hbm.at[i8sl], rows_v.at[slot, pl.ds(j * GSZ, GSZ)],
                    sem_v.at[slot])
                start = cb + j * GSZ
                if not isinstance(start, int):
                    start = pl.multiple_of(start, 8)
                isl = idx_v.at[pl.ds(start, GSZ)]
                pltpu.async_copy(
                    w_hbm.at[isl], wv_v.at[slot, pl.ds(j * GSZ, GSZ)],
                    sem_w.at[slot])

        def drain(slot):
            for j in range(NSTR):
                i8sl = idx8_v.at[slot, pl.ds(j * GSZ, GSZ)]
                pltpu.make_async_copy(
                    v8_hbm.at[i8sl], rows_v.at[slot, pl.ds(j * GSZ, GSZ)],
                    sem_v.at[slot]).wait()
                isl = idx_v.at[pl.ds(j * GSZ, GSZ)]
                pltpu.make_async_copy(
                    w_hbm.at[isl], wv_v.at[slot, pl.ds(j * GSZ, GSZ)],
                    sem_w.at[slot]).wait()

        def compute(g, slot):
            rows_g = rows_v.at[slot]
            wv_g = wv_v.at[slot]
            cb = _cb(g)

            lin = plsc.load_gather(wv_g, [lane_f])
            for f in range(1, F):
                lin = lin + plsc.load_gather(wv_g, [lane_f + f])
            oc = g * C if isinstance(g, int) else pl.multiple_of(g * C, C)
            out_v[pl.ds(oc, LANES)] = lin

            def pair_body(s, _):
                rb = s * F

                def row(f):
                    iv = plsc.load_gather(
                        idx_v, [jnp.broadcast_to(cb + rb + f, (LANES,))])
                    lanes = (iv & 7) * LANES + lane
                    rvec = jnp.broadcast_to(rb + f, (LANES,))
                    return plsc.load_gather(rows_g, [rvec, lanes])

                r = row(0)
                acc = r
                acc2 = r * r
                for f in range(1, F):
                    r = row(f)
                    acc = acc + r
                    acc2 = acc2 + r * r
                t = acc * acc - acc2
                cum = plsc.cumsum(t) * 0.5
                pos = jnp.broadcast_to(g * C + s, (LANES,)).astype(jnp.int32)
                plsc.addupdate_scatter(out_v, [pos], cum, mask=last)
                return 0

            lax.fori_loop(0, C, pair_body, 0, unroll=False)

        prep_and_fire(0, 0)

        def chunk_pair(g0, _):
            for b in range(2):
                g = g0 + b
                other = 1 - b

                @pl.when(g + 1 < NCH)
                def _():
                    prep_and_fire(g + 1, other)

                drain(b)
                compute(g, b)
            return 0

        lax.fori_loop(0, NCH // 2, lambda i, c: chunk_pair(i * 2, c), 0,
                      unroll=False)

        pltpu.sync_copy(out_v, out_hbm.at[pl.ds(wid * S, S)])

    return fm


def kernel(idx, w0, w, V):
    B, F = idx.shape
    N, K = V.shape
    v8 = V.reshape(N * K // 128, 128)
    out = _build(B, F, N, K)(idx.reshape(-1), w.reshape(-1), v8)
    return out + w0[0]


# speedup vs baseline: 1.0786x; 1.0786x over previous
import functools

import jax
import jax.numpy as jnp
from jax import lax
from jax.experimental import pallas as pl
from jax.experimental.pallas import tpu as pltpu
from jax.experimental.pallas import tpu_sc as plsc

NC = 2
NS = 16
NW = NC * NS
LANES = 16


def _mesh():
    return plsc.VectorSubcoreMesh(
        core_axis_name="c", subcore_axis_name="s",
        num_cores=NC, num_subcores=NS)


@functools.lru_cache(maxsize=None)
def _tc_repack(N, K):
    BK = 8192
    BT = BK // 8
    NT = N // 8
    assert N % BK == 0 and 128 % K == 0

    def body(vt_ref, out_ref, buf):
        buf[...] = vt_ref[...].T
        br = buf.reshape(BT, 8, K)
        out_ref[...] = jnp.concatenate(
            [br[:, m, :] for m in range(8)], axis=1)

    return pl.pallas_call(
        body,
        grid=(N // BK,),
        in_specs=[pl.BlockSpec((K, BK), lambda i: (0, i))],
        out_specs=pl.BlockSpec((BT, 128), lambda i: (i, 0)),
        out_shape=jax.ShapeDtypeStruct((NT, 128), jnp.float32),
        scratch_shapes=[pltpu.VMEM((BK, K), jnp.float32)],
    )


@functools.lru_cache(maxsize=None)
def _build(B, F, N, K, interpret=False):
    assert K == LANES
    assert B % NW == 0
    S = B // NW
    C = 64 if S % 64 == 0 else S
    NCH = S // C
    RPC = C * F
    GSZ = 128
    while RPC % GSZ:
        GSZ //= 2
    NSTR = RPC // GSZ
    NBUF = 2 if NCH > 1 else 1

    @functools.partial(
        pl.kernel,
        out_type=jax.ShapeDtypeStruct((B,), jnp.float32),
        mesh=_mesh(),
        scratch_types=[
            pltpu.VMEM((S * F,), jnp.int32),
            pltpu.VMEM((NBUF, RPC, K), jnp.float32),
            pltpu.VMEM((NBUF, RPC), jnp.float32),
            pltpu.VMEM((S,), jnp.float32),
            pltpu.SemaphoreType.DMA((NBUF,)),
            pltpu.SemaphoreType.DMA((NBUF,)),
        ],
        compiler_params=pltpu.CompilerParams(
            needs_layout_passes=False, use_tc_tiling_on_sc=False),
        interpret=interpret,
    )
    def fm(idx_hbm, w_hbm, v_hbm, out_hbm, idx_v, rows_v, wv_v, out_v,
           sem_v, sem_w):
        wid = lax.axis_index("s") * NC + lax.axis_index("c")
        base = wid * (S * F)
        pltpu.sync_copy(idx_hbm.at[pl.ds(base, S * F)], idx_v)

        lane = lax.iota(jnp.int32, LANES)
        lane_f = lane * F
        last = lane == (LANES - 1)

        def fire(g):
            slot = g % NBUF
            cps = []
            for j in range(NSTR):
                isl = idx_v.at[pl.ds(g * RPC + j * GSZ, GSZ)]
                cps.append(pltpu.async_copy(
                    v_hbm.at[isl], rows_v.at[slot, pl.ds(j * GSZ, GSZ)],
                    sem_v.at[slot]))
                cps.append(pltpu.async_copy(
                    w_hbm.at[isl], wv_v.at[slot, pl.ds(j * GSZ, GSZ)],
                    sem_w.at[slot]))
            return cps

        pending = {0: fire(0)}
        for g in range(NCH):
            if g + 1 < NCH:
                pending[g + 1] = fire(g + 1)
            for cp in pending.pop(g):
                cp.wait()
            slot = g % NBUF
            rows_g = rows_v.at[slot]
            wv_g = wv_v.at[slot]

            def lin_body(gg, _):
                sbase = lane_f + gg * (LANES * F)
                lin = plsc.load_gather(wv_g, [sbase])
                for f in range(1, F):
                    lin = lin + plsc.load_gather(wv_g, [sbase + f])
                out_v[pl.ds(g * C + gg * LANES, LANES)] = lin
                return 0

            lax.fori_loop(0, C // LANES, lin_body, 0, unroll=False)

            def pair_body(s, _):
                rb = s * F
                r = rows_g[rb, :]
                acc = r
                acc2 = r * r
                for f in range(1, F):
                    r = rows_g[rb + f, :]
                    acc = acc + r
                    acc2 = acc2 + r * r
                t = acc * acc - acc2
                cum = plsc.cumsum(t) * 0.5
                pos = jnp.broadcast_to(g * C + s, (LANES,)).astype(jnp.int32)
                plsc.addupdate_scatter(out_v, [pos], cum, mask=last)
                return 0

            lax.fori_loop(0, C, pair_body, 0, unroll=False)

        pltpu.sync_copy(out_v, out_hbm.at[pl.ds(wid * S, S)])

    return fm


def kernel(idx, w0, w, V):
    B, F = idx.shape
    N, K = V.shape
    NP = -(-N // 8192) * 8192
    vtp = jnp.pad(V.T, ((0, 0), (0, NP - N)))
    v1 = _tc_repack(NP, K)(vtp)
    out = _build(B, F, N, K)(idx.reshape(-1), w.reshape(-1), v1.reshape(NP, K))
    return out + w0[0]


# speedup vs baseline: 1.2084x; 1.1203x over previous
import functools

import jax
import jax.numpy as jnp
from jax import lax
from jax.experimental import pallas as pl
from jax.experimental.pallas import tpu as pltpu
from jax.experimental.pallas import tpu_sc as plsc

NC = 2
NS = 16
NW = NC * NS
LANES = 16


def _mesh():
    return plsc.VectorSubcoreMesh(
        core_axis_name="c", subcore_axis_name="s",
        num_cores=NC, num_subcores=NS)


@functools.lru_cache(maxsize=None)
def _tc_repack(N, K):
    BK = 8192
    BT = BK // 8
    NT = N // 8
    assert N % BK == 0 and 128 % K == 0

    def body(vt_ref, out_ref, buf):
        br = vt_ref[...].T.reshape(BT, 8, K)
        out_ref[...] = jnp.concatenate(
            [br[:, m, :] for m in range(8)], axis=1)

    return pl.pallas_call(
        body,
        grid=(N // BK,),
        in_specs=[pl.BlockSpec((K, BK), lambda i: (0, i))],
        out_specs=pl.BlockSpec((BT, 128), lambda i: (i, 0)),
        out_shape=jax.ShapeDtypeStruct((NT, 128), jnp.float32),
        scratch_shapes=[pltpu.VMEM((BK, K), jnp.float32)],
    )


@functools.lru_cache(maxsize=None)
def _build(B, F, N, K, interpret=False):
    assert K == LANES
    assert B % NW == 0
    S = B // NW
    C = 64 if S % 64 == 0 else S
    NCH = S // C
    RPC = C * F
    GSZ = 128
    while RPC % GSZ:
        GSZ //= 2
    NSTR = RPC // GSZ
    NBUF = 2 if NCH > 1 else 1

    @functools.partial(
        pl.kernel,
        out_type=jax.ShapeDtypeStruct((B,), jnp.float32),
        mesh=_mesh(),
        scratch_types=[
            pltpu.VMEM((S * F,), jnp.int32),
            pltpu.VMEM((NBUF, RPC, K), jnp.float32),
            pltpu.VMEM((NBUF, RPC), jnp.float32),
            pltpu.VMEM((S,), jnp.float32),
            pltpu.SemaphoreType.DMA((NBUF,)),
            pltpu.SemaphoreType.DMA((NBUF,)),
        ],
        compiler_params=pltpu.CompilerParams(
            needs_layout_passes=False, use_tc_tiling_on_sc=False),
        interpret=interpret,
    )
    def fm(idx_hbm, w_hbm, v_hbm, out_hbm, idx_v, rows_v, wv_v, out_v,
           sem_v, sem_w):
        wid = lax.axis_index("s") * NC + lax.axis_index("c")
        base = wid * (S * F)
        pltpu.sync_copy(idx_hbm.at[pl.ds(base, S * F)], idx_v)

        lane = lax.iota(jnp.int32, LANES)
        lane_f = lane * F
        last = lane == (LANES - 1)

        def fire(g):
            slot = g % NBUF
            cps = []
            for j in range(NSTR):
                isl = idx_v.at[pl.ds(g * RPC + j * GSZ, GSZ)]
                cps.append(pltpu.async_copy(
                    v_hbm.at[isl], rows_v.at[slot, pl.ds(j * GSZ, GSZ)],
                    sem_v.at[slot]))
                cps.append(pltpu.async_copy(
                    w_hbm.at[isl], wv_v.at[slot, pl.ds(j * GSZ, GSZ)],
                    sem_w.at[slot]))
            return cps

        pending = {0: fire(0)}
        for g in range(NCH):
            if g + 1 < NCH:
                pending[g + 1] = fire(g + 1)
            for cp in pending.pop(g):
                cp.wait()
            slot = g % NBUF
            rows_g = rows_v.at[slot]
            wv_g = wv_v.at[slot]

            def lin_body(gg, _):
                sbase = lane_f + gg * (LANES * F)
                lin = plsc.load_gather(wv_g, [sbase])
                for f in range(1, F):
                    lin = lin + plsc.load_gather(wv_g, [sbase + f])
                out_v[pl.ds(g * C + gg * LANES, LANES)] = lin
                return 0

            lax.fori_loop(0, C // LANES, lin_body, 0, unroll=False)

            def pair_body(s, _):
                rb = s * F
                r = rows_g[rb, :]
                acc = r
                acc2 = r * r
                for f in range(1, F):
                    r = rows_g[rb + f, :]
                    acc = acc + r
                    acc2 = acc2 + r * r
                t = acc * acc - acc2
                cum = plsc.cumsum(t) * 0.5
                pos = jnp.broadcast_to(g * C + s, (LANES,)).astype(jnp.int32)
                plsc.addupdate_scatter(out_v, [pos], cum, mask=last)
                return 0

            lax.fori_loop(0, C, pair_body, 0, unroll=False)

        pltpu.sync_copy(out_v, out_hbm.at[pl.ds(wid * S, S)])

    return fm


def kernel(idx, w0, w, V):
    B, F = idx.shape
    N, K = V.shape
    NP = -(-N // 8192) * 8192
    vtp = jnp.pad(V.T, ((0, 0), (0, NP - N)))
    v1 = _tc_repack(NP, K)(vtp)
    out = _build(B, F, N, K)(idx.reshape(-1), w.reshape(-1), v1.reshape(NP, K))
    return out + w0[0]


# speedup vs baseline: 1.3050x; 1.0799x over previous
import functools

import jax
import jax.numpy as jnp
from jax import lax
from jax.experimental import pallas as pl
from jax.experimental.pallas import tpu as pltpu
from jax.experimental.pallas import tpu_sc as plsc

NC = 2
NS = 16
NW = NC * NS
LANES = 16


def _mesh():
    return plsc.VectorSubcoreMesh(
        core_axis_name="c", subcore_axis_name="s",
        num_cores=NC, num_subcores=NS)


@functools.lru_cache(maxsize=None)
def _tc_repack(N, K):
    BK = 8192
    BT = BK // 8
    NT = N // 8
    assert N % BK == 0 and 128 % K == 0

    def body(vt_ref, wt_ref, out_ref, wout_ref):
        br = vt_ref[...].T.reshape(BT, 8, K)
        out_ref[...] = jnp.concatenate(
            [br[:, m, :] for m in range(8)], axis=1)
        wout_ref[...] = wt_ref[0, :]

    return pl.pallas_call(
        body,
        grid=(N // BK,),
        in_specs=[pl.BlockSpec((K, BK), lambda i: (0, i)),
                  pl.BlockSpec((1, BK), lambda i: (0, i))],
        out_specs=[pl.BlockSpec((BT, 128), lambda i: (i, 0)),
                   pl.BlockSpec((BK,), lambda i: (i,))],
        out_shape=(jax.ShapeDtypeStruct((NT, 128), jnp.float32),
                   jax.ShapeDtypeStruct((N,), jnp.float32)),
    )


@functools.lru_cache(maxsize=None)
def _build(B, F, N, K, interpret=False):
    assert K == LANES
    assert B % NW == 0
    S = B // NW
    C = 64 if S % 64 == 0 else S
    NCH = S // C
    RPC = C * F
    GSZ = 128
    while RPC % GSZ:
        GSZ //= 2
    NSTR = RPC // GSZ
    NBUF = 2 if NCH > 1 else 1

    @functools.partial(
        pl.kernel,
        out_type=jax.ShapeDtypeStruct((B,), jnp.float32),
        mesh=_mesh(),
        scratch_types=[
            pltpu.VMEM((S * F,), jnp.int32),
            pltpu.VMEM((NBUF, RPC, K), jnp.float32),
            pltpu.VMEM((NBUF, RPC), jnp.float32),
            pltpu.VMEM((S,), jnp.float32),
            pltpu.SemaphoreType.DMA((NBUF,)),
            pltpu.SemaphoreType.DMA((NBUF,)),
        ],
        compiler_params=pltpu.CompilerParams(
            needs_layout_passes=False, use_tc_tiling_on_sc=False),
        interpret=interpret,
    )
    def fm(idx_hbm, w_hbm, v_hbm, out_hbm, idx_v, rows_v, wv_v, out_v,
           sem_v, sem_w):
        wid = lax.axis_index("s") * NC + lax.axis_index("c")
        base = wid * (S * F)
        pltpu.sync_copy(idx_hbm.at[pl.ds(base, S * F)], idx_v)

        lane = lax.iota(jnp.int32, LANES)
        lane_f = lane * F
        last = lane == (LANES - 1)

        def fire(g):
            slot = g % NBUF
            cps = []
            for j in range(NSTR):
                isl = idx_v.at[pl.ds(g * RPC + j * GSZ, GSZ)]
                cps.append(pltpu.async_copy(
                    v_hbm.at[isl], rows_v.at[slot, pl.ds(j * GSZ, GSZ)],
                    sem_v.at[slot]))
                cps.append(pltpu.async_copy(
                    w_hbm.at[isl], wv_v.at[slot, pl.ds(j * GSZ, GSZ)],
                    sem_w.at[slot]))
            return cps

        pending = {0: fire(0)}
        for g in range(NCH):
            if g + 1 < NCH:
                pending[g + 1] = fire(g + 1)
            for cp in pending.pop(g):
                cp.wait()
            slot = g % NBUF
            rows_g = rows_v.at[slot]
            wv_g = wv_v.at[slot]

            def lin_body(gg, _):
                sbase = lane_f + gg * (LANES * F)
                lin = plsc.load_gather(wv_g, [sbase])
                for f in range(1, F):
                    lin = lin + plsc.load_gather(wv_g, [sbase + f])
                out_v[pl.ds(g * C + gg * LANES, LANES)] = lin
                return 0

            lax.fori_loop(0, C // LANES, lin_body, 0, unroll=False)

            def pair_body(s, _):
                rb = s * F
                r = rows_g[rb, :]
                acc = r
                acc2 = r * r
                for f in range(1, F):
                    r = rows_g[rb + f, :]
                    acc = acc + r
                    acc2 = acc2 + r * r
                t = acc * acc - acc2
                cum = plsc.cumsum(t) * 0.5
                pos = jnp.broadcast_to(g * C + s, (LANES,)).astype(jnp.int32)
                plsc.addupdate_scatter(out_v, [pos], cum, mask=last)
                return 0

            lax.fori_loop(0, C, pair_body, 0, unroll=False)

        pltpu.sync_copy(out_v, out_hbm.at[pl.ds(wid * S, S)])

    return fm


def kernel(idx, w0, w, V):
    B, F = idx.shape
    N, K = V.shape
    NP = -(-N // 8192) * 8192
    vtp = jnp.pad(V.T, ((0, 0), (0, NP - N)))
    wtp = jnp.pad(w.T, ((0, 0), (0, NP - N)))
    v1, w1 = _tc_repack(NP, K)(vtp, wtp)
    out = _build(B, F, N, K)(idx.reshape(-1), w1, v1.reshape(NP, K))
    return out + w0[0]
